# baseline (speedup 1.0000x reference)
import functools

import jax
import jax.numpy as jnp
from jax import lax
from jax.experimental import pallas as pl
from jax.experimental.pallas import tpu as pltpu
from jax.experimental.pallas import tpu_sc as plsc

_EPS = 1e-5
_NC = 2
_NS = 16
_NW = _NC * _NS
_CHUNK = 128


def _sc_gather(table, idx3d, latent):
    chunks_per_w = idx3d.shape[1]
    n_idx = _NW * chunks_per_w * _CHUNK
    mesh = plsc.VectorSubcoreMesh(core_axis_name="c", subcore_axis_name="s")

    @functools.partial(
        pl.kernel,
        mesh=mesh,
        out_type=jax.ShapeDtypeStruct((n_idx, latent), jnp.float32),
        scratch_types=[
            pltpu.VMEM((chunks_per_w, _CHUNK), jnp.int32),
            pltpu.VMEM((_CHUNK, latent), jnp.float32),
            pltpu.SemaphoreType.DMA,
        ],
        compiler_params=pltpu.CompilerParams(use_tc_tiling_on_sc=False),
    )
    def k(table_hbm, idx_hbm, out_hbm, idx_v, rows_v, sem):
        wid = lax.axis_index("s") * _NC + lax.axis_index("c")
        crow0 = wid * chunks_per_w
        pltpu.sync_copy(idx_hbm.at[wid], idx_v)

        def body(j, carry):
            pltpu.async_copy(table_hbm.at[idx_v.at[j]], rows_v, sem).wait()
            pltpu.sync_copy(rows_v, out_hbm.at[pl.ds((crow0 + j) * _CHUNK,
                                                     _CHUNK)])
            return carry

        lax.fori_loop(0, chunks_per_w, body, 0)

    return k(table, idx3d)


def _tc_half(seq_rep, aa2w, Wst, bs, Wtt, bt, gs, bes, gt, bet,
             phase, prev):
    B, L, D = seq_rep.shape
    latent = aa2w.shape[-1] // 2
    H = L // 2
    bB = 32
    half_blocks = (B // 2) // bB
    grid = (half_blocks,)
    off = phase * half_blocks

    def body(seq_ref, aa_ref, wst_ref, bs_ref, wtt_ref, bt_ref,
             gs_ref, bes_ref, gt_ref, bet_ref, *rest):
        out_ref = rest[-1]
        x = seq_ref[...]
        m = jnp.mean(x, axis=-1, keepdims=True)
        xc = x - m
        v = jnp.mean(xc * xc, axis=-1, keepdims=True)
        xn = xc * lax.rsqrt(v + _EPS)
        xn = xn * gt_ref[...].reshape(1, 1, D) + bet_ref[...].reshape(1, 1, D)
        xp = xn.reshape(bB, H, 2, D)
        w_tok = wtt_ref[...]
        b_tok = bt_ref[...].reshape(1, 1, latent)
        tokE = jnp.dot(xp[:, :, 0, :].reshape(bB * H, D), w_tok,
                       preferred_element_type=jnp.float32).reshape(bB, H, latent)
        tokO = jnp.dot(xp[:, :, 1, :].reshape(bB * H, D), w_tok,
                       preferred_element_type=jnp.float32).reshape(bB, H, latent)
        sm = jnp.mean(x, axis=1)
        m2 = jnp.mean(sm, axis=-1, keepdims=True)
        sc = sm - m2
        v2 = jnp.mean(sc * sc, axis=-1, keepdims=True)
        sn = sc * lax.rsqrt(v2 + _EPS) * gs_ref[...] + bes_ref[...]
        se = jnp.dot(sn, wst_ref[...], preferred_element_type=jnp.float32)
        se = (se + bs_ref[...]).reshape(bB, 1, latent)
        aa = aa_ref[...].reshape(bB, H, 2 * latent)
        addE = tokE + b_tok + se + aa[:, :, :latent]
        addO = tokO + b_tok + se + aa[:, :, latent:]
        out_ref[...] = jnp.stack([addE, addO], axis=2).reshape(bB, L, latent)

    in_specs = [
        pl.BlockSpec((bB, L, D), lambda i: (i + off, 0, 0)),
        pl.BlockSpec((bB * H, 2 * latent), lambda i: (i, 0)),
        pl.BlockSpec((D, latent), lambda i: (0, 0)),
        pl.BlockSpec((1, latent), lambda i: (0, 0)),
        pl.BlockSpec((D, latent), lambda i: (0, 0)),
        pl.BlockSpec((1, latent), lambda i: (0, 0)),
        pl.BlockSpec((1, D), lambda i: (0, 0)),
        pl.BlockSpec((1, D), lambda i: (0, 0)),
        pl.BlockSpec((1, D), lambda i: (0, 0)),
        pl.BlockSpec((1, D), lambda i: (0, 0)),
    ]
    args = [seq_rep, aa2w, Wst, bs, Wtt, bt, gs, bes, gt, bet]
    aliases = {}
    if prev is not None:
        in_specs.append(pl.BlockSpec((1, 8, latent), lambda i: (0, 0, 0)))
        args.append(prev)
        aliases = {len(args) - 1: 0}

    return pl.pallas_call(
        body,
        grid=grid,
        in_specs=in_specs,
        out_specs=pl.BlockSpec((bB, L, latent), lambda i: (i + off, 0, 0)),
        out_shape=jax.ShapeDtypeStruct((B, L, latent), jnp.float32),
        input_output_aliases=aliases,
    )(*args)


def kernel(aa_types, seq_rep, aa_table, W_seq, b_seq, W_tok, b_tok,
           g_seq, be_seq, g_tok, be_tok):
    B, L, D = seq_rep.shape
    latent = aa_table.shape[-1]
    half = B // 2
    idx = aa_types.astype(jnp.int32)
    idx_a = idx[:half].reshape(_NW, -1, _CHUNK)
    idx_b = idx[half:].reshape(_NW, -1, _CHUNK)
    g1 = _sc_gather(aa_table, idx_a, latent)
    g2 = _sc_gather(aa_table, idx_b, latent)
    aa1 = g1.reshape(half * L // 2, 2 * latent)
    aa2 = g2.reshape(half * L // 2, 2 * latent)
    params = (W_seq.T, b_seq.reshape(1, -1), W_tok.T, b_tok.reshape(1, -1),
              g_seq.reshape(1, -1), be_seq.reshape(1, -1),
              g_tok.reshape(1, -1), be_tok.reshape(1, -1))
    out1 = _tc_half(seq_rep, aa1, *params, phase=0, prev=None)
    out = _tc_half(seq_rep, aa2, *params, phase=1, prev=out1)
    return out


# speedup vs baseline: 1.2672x; 1.2672x over previous
import functools

import jax
import jax.numpy as jnp
from jax import lax
from jax.experimental import pallas as pl
from jax.experimental.pallas import tpu as pltpu
from jax.experimental.pallas import tpu_sc as plsc

_EPS = 1e-5
_NC = 2
_NS = 16
_NW = _NC * _NS
_CHUNK = 128


def _sc_gather(table, idx2d, latent):
    Bb, L = idx2d.shape
    rows_per_w = Bb // _NW
    c1 = min(L, _CHUNK)
    c2 = L - c1
    n_idx = Bb * L
    mesh = plsc.VectorSubcoreMesh(core_axis_name="c", subcore_axis_name="s")

    @functools.partial(
        pl.kernel,
        mesh=mesh,
        out_type=jax.ShapeDtypeStruct((n_idx, latent), jnp.float32),
        scratch_types=[
            pltpu.VMEM((rows_per_w, L), jnp.int32),
            pltpu.VMEM((2, c1, latent), jnp.float32),
            pltpu.VMEM((2, c2, latent), jnp.float32),
            pltpu.SemaphoreType.DMA((2,)),
            pltpu.SemaphoreType.DMA((2,)),
        ],
        compiler_params=pltpu.CompilerParams(use_tc_tiling_on_sc=False),
    )
    def k(table_hbm, idx_hbm, out_hbm, idx_v, bufA, bufB, semA, semB):
        wid = lax.axis_index("s") * _NC + lax.axis_index("c")
        row0 = wid * rows_per_w
        base = row0 * L
        pltpu.sync_copy(idx_hbm.at[pl.ds(row0, rows_per_w)], idx_v)

        def start(r, slot):
            pltpu.async_copy(table_hbm.at[idx_v.at[r, pl.ds(0, c1)]],
                             bufA.at[slot], semA.at[slot])
            pltpu.async_copy(table_hbm.at[idx_v.at[r, pl.ds(c1, c2)]],
                             bufB.at[slot], semB.at[slot])

        def wait(r, slot):
            pltpu.make_async_copy(table_hbm.at[idx_v.at[r, pl.ds(0, c1)]],
                                  bufA.at[slot], semA.at[slot]).wait()
            pltpu.make_async_copy(table_hbm.at[idx_v.at[r, pl.ds(c1, c2)]],
                                  bufB.at[slot], semB.at[slot]).wait()

        start(0, 0)

        def body(r, carry):
            slot = lax.rem(r, 2)

            @pl.when(r + 1 < rows_per_w)
            def _():
                start(r + 1, lax.rem(r + 1, 2))

            wait(r, slot)
            pltpu.sync_copy(bufA.at[slot], out_hbm.at[pl.ds(base + r * L, c1)])
            pltpu.sync_copy(bufB.at[slot],
                            out_hbm.at[pl.ds(base + r * L + c1, c2)])
            return carry

        lax.fori_loop(0, rows_per_w, body, 0)

    return k(table, idx2d)


def _tc_dense(seq_rep, aa2w, Wst, bs, Wtt, bt, gs, bes, gt, bet):
    B, L, D = seq_rep.shape
    latent = aa2w.shape[-1] // 2
    H = L // 2
    bB = 32
    grid = (B // bB,)

    def body(seq_ref, aa_ref, wst_ref, bs_ref, wtt_ref, bt_ref,
             gs_ref, bes_ref, gt_ref, bet_ref, out_ref):
        x = seq_ref[...]
        m = jnp.mean(x, axis=-1, keepdims=True)
        xc = x - m
        v = jnp.mean(xc * xc, axis=-1, keepdims=True)
        xn = xc * lax.rsqrt(v + _EPS)
        xn = xn * gt_ref[...].reshape(1, 1, D) + bet_ref[...].reshape(1, 1, D)
        xp = xn.reshape(bB, H, 2, D)
        w_tok = wtt_ref[...]
        b_tok = bt_ref[...].reshape(1, 1, latent)
        tokE = jnp.dot(xp[:, :, 0, :].reshape(bB * H, D), w_tok,
                       preferred_element_type=jnp.float32).reshape(bB, H, latent)
        tokO = jnp.dot(xp[:, :, 1, :].reshape(bB * H, D), w_tok,
                       preferred_element_type=jnp.float32).reshape(bB, H, latent)
        sm = jnp.mean(x, axis=1)
        m2 = jnp.mean(sm, axis=-1, keepdims=True)
        sc = sm - m2
        v2 = jnp.mean(sc * sc, axis=-1, keepdims=True)
        sn = sc * lax.rsqrt(v2 + _EPS) * gs_ref[...] + bes_ref[...]
        se = jnp.dot(sn, wst_ref[...], preferred_element_type=jnp.float32)
        se = (se + bs_ref[...]).reshape(bB, 1, latent)
        aa = aa_ref[...].reshape(bB, H, 2 * latent)
        addE = tokE + b_tok + se + aa[:, :, :latent]
        addO = tokO + b_tok + se + aa[:, :, latent:]
        out_ref[...] = jnp.concatenate([addE, addO], axis=-1)

    return pl.pallas_call(
        body,
        grid=grid,
        in_specs=[
            pl.BlockSpec((bB, L, D), lambda i: (i, 0, 0)),
            pl.BlockSpec((bB * H, 2 * latent), lambda i: (i, 0)),
            pl.BlockSpec((D, latent), lambda i: (0, 0)),
            pl.BlockSpec((1, latent), lambda i: (0, 0)),
            pl.BlockSpec((D, latent), lambda i: (0, 0)),
            pl.BlockSpec((1, latent), lambda i: (0, 0)),
            pl.BlockSpec((1, D), lambda i: (0, 0)),
            pl.BlockSpec((1, D), lambda i: (0, 0)),
            pl.BlockSpec((1, D), lambda i: (0, 0)),
            pl.BlockSpec((1, D), lambda i: (0, 0)),
        ],
        out_specs=pl.BlockSpec((bB, H, 2 * latent), lambda i: (i, 0, 0)),
        out_shape=jax.ShapeDtypeStruct((B, H, 2 * latent), jnp.float32),
    )(seq_rep, aa2w, Wst, bs, Wtt, bt, gs, bes, gt, bet)


def kernel(aa_types, seq_rep, aa_table, W_seq, b_seq, W_tok, b_tok,
           g_seq, be_seq, g_tok, be_tok):
    B, L, D = seq_rep.shape
    latent = aa_table.shape[-1]
    aa_flat = _sc_gather(aa_table, aa_types.astype(jnp.int32), latent)
    aa2w = aa_flat.reshape(B * L // 2, 2 * latent)
    out2w = _tc_dense(
        seq_rep, aa2w,
        W_seq.T, b_seq.reshape(1, -1),
        W_tok.T, b_tok.reshape(1, -1),
        g_seq.reshape(1, -1), be_seq.reshape(1, -1),
        g_tok.reshape(1, -1), be_tok.reshape(1, -1),
    )
    return out2w.reshape(B, L, latent)
